# baseline (speedup 1.0000x reference)
import functools

import jax
import jax.numpy as jnp
from jax import lax
from jax.experimental import pallas as pl
from jax.experimental.pallas import tpu as pltpu
from jax.experimental.pallas import tpu_sc as plsc

_NUM_EMB = 1000000
_D = 32
_NW = 32
_L = 16
_NBUF = 6
_DEPTH = 4

_VFULL = _NUM_EMB // 128
_NSLOT = 246


@functools.lru_cache(maxsize=None)
def _build_tab():
    mesh = plsc.VectorSubcoreMesh(core_axis_name="c", subcore_axis_name="s")

    @functools.partial(
        pl.kernel,
        out_type=jax.ShapeDtypeStruct((_NUM_EMB * _D // 128, 128), jnp.float32),
        mesh=mesh,
        scratch_types=[
            *[pltpu.VMEM((_D, 128), jnp.float32) for _ in range(2)],
            *[pltpu.VMEM((_D, 128), jnp.float32) for _ in range(2)],
            pltpu.VMEM((_D, 64), jnp.float32),
            pltpu.VMEM((16, 128), jnp.float32),
            *[pltpu.SemaphoreType.DMA for _ in range(4)],
        ],
        compiler_params=pltpu.CompilerParams(
            use_tc_tiling_on_sc=True, needs_layout_passes=False, disable_bounds_checks=True
        ),
    )
    def ktab(tt_hbm, r_hbm, *rest):
        src = rest[0:2]
        dst = rest[2:4]
        src64, dst16 = rest[4], rest[5]
        isems = rest[6:8]
        osems = rest[8:10]

        wid = lax.axis_index("s") * 2 + lax.axis_index("c")
        lane = lax.iota(jnp.int32, 16)
        kpat = lane // 4
        jpat = (lane % 4) * _D
        kvecs = [kpat + 4 * g for g in range(8)]

        def bc_of(ip):
            return jnp.minimum(ip * _NW + wid, _VFULL - 1)

        def fire_in(ip, b):
            bc = bc_of(ip)
            pltpu.async_copy(
                tt_hbm.at[:, pl.ds(bc * 128, 128)], src[b], isems[b]
            )

        def wait_in(b):
            pltpu.make_async_copy(
                tt_hbm.at[:, pl.ds(0, 128)], src[b], isems[b]
            ).wait()

        def fire_out(ip, b):
            bc = bc_of(ip)
            pltpu.async_copy(dst[b], r_hbm.at[pl.ds(bc * _D, _D)], osems[b])

        def wait_out(b):
            pltpu.make_async_copy(
                dst[b], r_hbm.at[pl.ds(0, _D)], osems[b]
            ).wait()

        def transpose(b):
            for d in range(_D):
                idx_j = jpat + d
                for g in range(8):
                    v = src[b][d, pl.ds(16 * g, 16)]
                    plsc.store_scatter(dst[b], [kvecs[g], idx_j], v)

        fire_in(0, 0)
        fire_in(1, 1)
        for b in (0, 1):
            wait_in(b)
            transpose(b)
            fire_out(b, b)
            fire_in(b + 2, b)

        def round_body(p, carry):
            for b in (0, 1):
                ip = 2 * p + b
                wait_in(b)
                wait_out(b)
                transpose(b)
                fire_out(ip, b)

                @pl.when(ip + 2 < _NSLOT)
                def _():
                    fire_in(ip + 2, b)

            return carry

        lax.fori_loop(1, _NSLOT // 2, round_body, 0)
        for b in (0, 1):
            wait_out(b)

        @pl.when(wid == _NW - 1)
        def _():
            pltpu.sync_copy(tt_hbm.at[:, pl.ds(_VFULL * 128, 64)], src64)
            for d in range(_D):
                idx_j = jpat + d
                for g in range(4):
                    v = src64[d, pl.ds(16 * g, 16)]
                    plsc.store_scatter(dst16, [kvecs[g], idx_j], v)
            pltpu.sync_copy(dst16, r_hbm.at[pl.ds(_VFULL * _D, 16)])

    return ktab


@functools.lru_cache(maxsize=None)
def _build_permute(T, J):
    TPW = T // _NW
    BPW = TPW * J
    mesh = plsc.VectorSubcoreMesh(core_axis_name="c", subcore_axis_name="s")

    @functools.partial(
        pl.kernel,
        out_type=jax.ShapeDtypeStruct((T * J,), jnp.int32),
        mesh=mesh,
        scratch_types=[
            pltpu.VMEM((J, TPW), jnp.int32),
            pltpu.VMEM((BPW,), jnp.int32),
        ],
        compiler_params=pltpu.CompilerParams(
            use_tc_tiling_on_sc=True, needs_layout_passes=False, disable_bounds_checks=True
        ),
    )
    def k0(xt_hbm, xp_hbm, slab, flat):
        wid = lax.axis_index("s") * 2 + lax.axis_index("c")
        pltpu.sync_copy(xt_hbm.at[:, pl.ds(wid * TPW, TPW)], slab)

        def body(i, carry):
            j = i // (TPW // _L)
            g = i % (TPW // _L)
            v = slab[j, pl.ds(g * _L, _L)]
            v = jnp.minimum(jnp.maximum(v, 0), _NUM_EMB - 1)
            flat[pl.ds(j * TPW + g * _L, _L)] = v
            return carry

        lax.fori_loop(0, J * (TPW // _L), body, 0)
        pltpu.sync_copy(flat, xp_hbm.at[pl.ds(wid * BPW, BPW)])

    return k0


@functools.lru_cache(maxsize=None)
def _build_gather(T, J):
    B = T * J
    TPW = T // _NW
    BPW = B // _NW
    NCH = J
    assert TPW % _L == 0 and TPW % 8 == 0 and NCH > _NBUF

    mesh = plsc.VectorSubcoreMesh(core_axis_name="c", subcore_axis_name="s")

    @functools.partial(
        pl.kernel,
        out_type=jax.ShapeDtypeStruct((T, J, _D), jnp.float32),
        mesh=mesh,
        scratch_types=[
            pltpu.VMEM((BPW,), jnp.int32),
            *[pltpu.VMEM((TPW, _D), jnp.float32) for _ in range(_NBUF)],
            *[pltpu.SemaphoreType.DMA for _ in range(2 * _NBUF)],
        ],
        compiler_params=pltpu.CompilerParams(
            use_tc_tiling_on_sc=False, needs_layout_passes=False, disable_bounds_checks=True
        ),
    )
    def k1(x_hbm, tab_hbm, out_hbm, idx_v, *rest):
        bufs = rest[:_NBUF]
        gsems = rest[_NBUF:2 * _NBUF]
        wsems = rest[2 * _NBUF:]

        wid = lax.axis_index("s") * 2 + lax.axis_index("c")
        base = wid * BPW
        t_base = wid * TPW
        pltpu.sync_copy(x_hbm.at[pl.ds(base, BPW)], idx_v)

        gd = [None] * _NBUF
        wd = [None] * _NBUF

        def fire_gather(c):
            b = c % _NBUF
            gd[b] = pltpu.async_copy(
                tab_hbm.at[idx_v.at[pl.ds(c * TPW, TPW)]], bufs[b], gsems[b]
            )

        for j in range(_DEPTH):
            fire_gather(j)
        for c in range(NCH):
            b = c % _NBUF
            if c + _DEPTH < NCH:
                pb = (c + _DEPTH) % _NBUF
                if c + _DEPTH - _NBUF >= 0:
                    wd[pb].wait()
                fire_gather(c + _DEPTH)
            gd[b].wait()
            wd[b] = pltpu.async_copy(
                bufs[b], out_hbm.at[pl.ds(t_base, TPW), c], wsems[b]
            )
        for b in range(_NBUF):
            wd[b].wait()

    return k1


def kernel(x, embedding_table):
    T, J = x.shape
    tt = jnp.transpose(embedding_table)
    rr = _build_tab()(tt)
    rt = jnp.reshape(rr, (_NUM_EMB, _D))
    xp = _build_permute(T, J)(jnp.transpose(x).astype(jnp.int32))
    return _build_gather(T, J)(xp, rt)


# speedup vs baseline: 1.2158x; 1.2158x over previous
import functools

import jax
import jax.numpy as jnp
from jax import lax
from jax.experimental import pallas as pl
from jax.experimental.pallas import tpu as pltpu
from jax.experimental.pallas import tpu_sc as plsc

_NUM_EMB = 1000000
_D = 32
_NW = 32
_L = 16


@functools.lru_cache(maxsize=None)
def _build_permute(T, J):
    TPW = T // _NW
    BPW = TPW * J
    mesh = plsc.VectorSubcoreMesh(core_axis_name="c", subcore_axis_name="s")

    @functools.partial(
        pl.kernel,
        out_type=jax.ShapeDtypeStruct((T * J,), jnp.int32),
        mesh=mesh,
        scratch_types=[
            pltpu.VMEM((J, TPW), jnp.int32),
            pltpu.VMEM((BPW,), jnp.int32),
        ],
        compiler_params=pltpu.CompilerParams(
            use_tc_tiling_on_sc=True, needs_layout_passes=False
        ),
    )
    def k0(xt_hbm, xp_hbm, slab, flat):
        wid = lax.axis_index("s") * 2 + lax.axis_index("c")
        pltpu.sync_copy(xt_hbm.at[:, pl.ds(wid * TPW, TPW)], slab)

        def body(i, carry):
            j = i // (TPW // _L)
            g = i % (TPW // _L)
            v = slab[j, pl.ds(g * _L, _L)]
            v = jnp.minimum(jnp.maximum(v, 0), _NUM_EMB - 1)
            flat[pl.ds(j * TPW + g * _L, _L)] = v
            return carry

        lax.fori_loop(0, J * (TPW // _L), body, 0)
        pltpu.sync_copy(flat, xp_hbm.at[pl.ds(wid * BPW, BPW)])

    return k0


@functools.lru_cache(maxsize=None)
def _build_gather(T, J):
    B = T * J
    TPW = T // _NW
    BPW = B // _NW
    NCH = J
    CC = TPW // 128
    NR = _D // 8
    ZW = TPW * _D
    assert TPW % 128 == 0 and NCH % 2 == 0 and NCH >= 6

    mesh = plsc.VectorSubcoreMesh(core_axis_name="c", subcore_axis_name="s")

    @functools.partial(
        pl.kernel,
        out_type=jax.ShapeDtypeStruct((B * _D,), jnp.float32),
        mesh=mesh,
        scratch_types=[
            pltpu.VMEM((BPW,), jnp.int32),
            *[pltpu.VMEM((TPW, _D), jnp.float32) for _ in range(2)],
            *[pltpu.VMEM((ZW,), jnp.float32) for _ in range(2)],
            *[pltpu.SemaphoreType.DMA for _ in range(4)],
        ],
        compiler_params=pltpu.CompilerParams(
            use_tc_tiling_on_sc=False, needs_layout_passes=False
        ),
    )
    def k1(x_hbm, tab_hbm, out_hbm, idx_v, *rest):
        bufs = rest[0:2]
        zbufs = rest[2:4]
        gsems = rest[4:6]
        wsems = rest[6:8]

        wid = lax.axis_index("s") * 2 + lax.axis_index("c")
        base = wid * BPW
        pltpu.sync_copy(x_hbm.at[pl.ds(base, BPW)], idx_v)

        lane = lax.iota(jnp.int32, 16)
        q = (lane // 8) * (CC * 1024) + (lane % 8) * 128

        def fire_gather(c, b):
            pltpu.async_copy(
                tab_hbm.at[idx_v.at[pl.ds(c * TPW, TPW)]], bufs[b], gsems[b]
            )

        def wait_gather(b):
            pltpu.make_async_copy(
                tab_hbm.at[idx_v.at[pl.ds(0, TPW)]], bufs[b], gsems[b]
            ).wait()

        def fire_writes(c, b):
            for r in range(NR):
                off = ((c * NR + r) * (T // 128) + wid * CC) * 1024
                pltpu.async_copy(
                    zbufs[b].at[pl.ds(r * (CC * 1024), CC * 1024)],
                    out_hbm.at[pl.ds(off, CC * 1024)],
                    wsems[b],
                )

        def wait_writes(b):
            for r in range(NR):
                pltpu.make_async_copy(
                    zbufs[b].at[pl.ds(0, CC * 1024)],
                    out_hbm.at[pl.ds(0, CC * 1024)],
                    wsems[b],
                ).wait()

        def transpose_chunk(b):
            rows = bufs[b]
            z = zbufs[b]

            def outer(cc, carry):
                def inner(l, carry2):
                    t = cc * 128 + l
                    for h in range(2):
                        addr = q + (h * (2 * CC * 1024) + cc * 1024 + l)
                        v = rows[t, pl.ds(16 * h, 16)]
                        plsc.store_scatter(z, [addr], v)
                    return carry2

                lax.fori_loop(0, 128, inner, 0)
                return carry

            lax.fori_loop(0, CC, outer, 0)

        fire_gather(0, 0)
        fire_gather(1, 1)
        for c in (0, 1):
            wait_gather(c)
            transpose_chunk(c)
            fire_gather(c + 2, c)
            fire_writes(c, c)

        def round_body(p, carry):
            for b in (0, 1):
                c = 2 * p + b
                wait_gather(b)
                wait_writes(b)
                transpose_chunk(b)

                @pl.when(c + 2 < NCH)
                def _():
                    fire_gather(c + 2, b)

                fire_writes(c, b)
            return carry

        lax.fori_loop(1, NCH // 2, round_body, 0)
        for b in (0, 1):
            wait_writes(b)

    return k1


def kernel(x, embedding_table):
    T, J = x.shape
    xp = _build_permute(T, J)(jnp.transpose(x).astype(jnp.int32))
    rr = lax.optimization_barrier(
        jnp.reshape(embedding_table, (_NUM_EMB * _D // 128, 128))
    )
    rt = jnp.reshape(rr, (_NUM_EMB, _D))
    z = _build_gather(T, J)(xp, rt)
    z5 = jnp.reshape(z, (J, _D // 8, T // 128, 8, 128))
    return jnp.transpose(z5, (2, 4, 0, 1, 3)).reshape(T, J, _D)
